# Initial kernel scaffold; baseline (speedup 1.0000x reference)
import jax
import jax.numpy as jnp
from jax.experimental import pallas as pl


def kernel(encoded_paths, paths_mask, paths_node_indices, previous_nodes_encodings, nr_nodes, W_upd, b_upd, W_gate, b_gate):
    raise NotImplementedError("write your pallas kernel here")


# speedup vs baseline: 2.5647x; 2.5647x over previous
import functools

import jax
import jax.numpy as jnp
from jax import lax
from jax.experimental import pallas as pl
from jax.experimental.pallas import tpu as pltpu
from jax.experimental.pallas import tpu_sc as plsc

P, L, D, N = 75000, 8, 128, 100000
PL = P * L
NLANES = 16
NCOLCH = D // NLANES
NCORES = 2
NTILES = 16
NPASS = NCOLCH // NCORES
C = 128
G = 8
NCHUNKS = 296
TPOS = NCHUNKS * C
PLPAD = NTILES * TPOS
NG = NCHUNKS // G
T15_FULLG = 30
T15_TAILC = 7
DUMP = N
ZROWS = 100352
ZT = ZROWS // NTILES
ZB = 224
OT = N // NTILES


def _sc_scatter(vals3, tail3, idx3):
    mesh = plsc.VectorSubcoreMesh(core_axis_name="c", subcore_axis_name="s")

    @functools.partial(
        pl.kernel,
        out_type=jax.ShapeDtypeStruct((N, NCOLCH, NLANES), jnp.float32),
        mesh=mesh,
        compiler_params=pltpu.CompilerParams(use_tc_tiling_on_sc=False),
        scratch_types=[
            pltpu.VMEM((G, C), jnp.int32),
            pltpu.VMEM((G * C, 1, NLANES), jnp.float32),
            pltpu.VMEM((ZB, 1, NLANES), jnp.float32),
            pltpu.VMEM_SHARED((ZROWS, 1, NLANES), jnp.float32),
            pltpu.SemaphoreType.DMA((G,)),
            pltpu.SemaphoreType.DMA((G,)),
        ],
    )
    def k(vals_hbm, tail_hbm, idx_hbm, out_hbm,
          idxgrp, gbuf, zbuf, acc, gsem, ssem):
        c = lax.axis_index("c")
        s = lax.axis_index("s")
        is_last = s == NTILES - 1
        ngroups = jnp.where(is_last, T15_FULLG, NG)

        def zb_body(i, carry):
            zbuf[i, 0] = jnp.zeros((NLANES,), jnp.float32)
            return carry
        lax.fori_loop(0, ZB, zb_body, 0)

        def gather_desc(base0, kcol, j):
            return pltpu.make_async_copy(
                vals_hbm.at[pl.ds(base0 + j * C, C), pl.ds(kcol, 1)],
                gbuf.at[pl.ds(j * C, C)], gsem.at[j])

        def scatter_desc(j):
            return pltpu.make_async_copy(
                gbuf.at[pl.ds(j * C, C)], acc.at[idxgrp.at[j]], ssem.at[j])

        for p in range(NPASS):
            kcol = p * NCORES + c

            for j in range(ZT // ZB):
                pltpu.sync_copy(zbuf, acc.at[pl.ds(s * ZT + j * ZB, ZB)])
            plsc.subcore_barrier()

            def group_body(g, carry):
                @pl.when(g > 0)
                def _():
                    for j in range(G):
                        scatter_desc(j).wait()
                pltpu.sync_copy(idx_hbm.at[s].at[pl.ds(g * G, G)], idxgrp)
                base0 = s * TPOS + g * (G * C)
                for j in range(G):
                    gather_desc(base0, kcol, j).start()
                for j in range(G):
                    gather_desc(base0, kcol, j).wait()
                    scatter_desc(j).start(add=True)
                return carry
            lax.fori_loop(0, ngroups, group_body, 0)
            for j in range(G):
                scatter_desc(j).wait()

            @pl.when(is_last)
            def _():
                pltpu.sync_copy(
                    idx_hbm.at[s].at[pl.ds(T15_FULLG * G, G)], idxgrp)
                for j in range(T15_TAILC):
                    base = s * TPOS + (T15_FULLG * G + j) * C
                    pltpu.sync_copy(
                        vals_hbm.at[pl.ds(base, C), pl.ds(kcol, 1)],
                        gbuf.at[pl.ds(0, C)])
                    pltpu.sync_copy(gbuf.at[pl.ds(0, C)],
                                    acc.at[idxgrp.at[j]], add=True)
                pltpu.sync_copy(tail_hbm.at[pl.ds(0, C), pl.ds(kcol, 1)],
                                gbuf.at[pl.ds(0, C)])
                pltpu.sync_copy(gbuf.at[pl.ds(0, C)],
                                acc.at[idxgrp.at[G - 1]], add=True)

            plsc.subcore_barrier()

            pltpu.sync_copy(
                acc.at[pl.ds(s * OT, OT)],
                out_hbm.at[pl.ds(s * OT, OT), pl.ds(kcol, 1)],
            )
            plsc.subcore_barrier()

    return k(vals3, tail3, idx3)


def _tc_update(scattered, prev, W_upd, b_upd, Wg_hi, Wg_lo, b_gate):
    B = 1000
    grid = (N // B,)

    def body(s_ref, p_ref, wu_ref, bu_ref, wgh_ref, wgl_ref, bg_ref, o_ref):
        sblk = s_ref[...]
        prv = p_ref[...]
        upd = jnp.dot(sblk, wu_ref[...], preferred_element_type=jnp.float32)
        upd = jnp.maximum(upd + bu_ref[...], 0.0)
        z = (jnp.dot(prv, wgh_ref[...], preferred_element_type=jnp.float32)
             + jnp.dot(upd, wgl_ref[...], preferred_element_type=jnp.float32)
             + bg_ref[...])
        g = jax.nn.sigmoid(z)
        o_ref[...] = g * prv + (1.0 - g) * upd

    return pl.pallas_call(
        body,
        grid=grid,
        in_specs=[
            pl.BlockSpec((B, D), lambda i: (i, 0)),
            pl.BlockSpec((B, D), lambda i: (i, 0)),
            pl.BlockSpec((D, D), lambda i: (0, 0)),
            pl.BlockSpec((1, D), lambda i: (0, 0)),
            pl.BlockSpec((D, D), lambda i: (0, 0)),
            pl.BlockSpec((D, D), lambda i: (0, 0)),
            pl.BlockSpec((1, D), lambda i: (0, 0)),
        ],
        out_specs=pl.BlockSpec((B, D), lambda i: (i, 0)),
        out_shape=jax.ShapeDtypeStruct((N, D), jnp.float32),
    )(scattered, prev, W_upd, b_upd.reshape(1, D), Wg_hi, Wg_lo,
      b_gate.reshape(1, D))


def kernel(encoded_paths, paths_mask, paths_node_indices,
           previous_nodes_encodings, nr_nodes, W_upd, b_upd, W_gate, b_gate):
    vals = encoded_paths.reshape(PL, D)
    vals3 = vals.reshape(PL, NCOLCH, NLANES)
    tail3 = jnp.concatenate(
        [vals[PL - 64:], jnp.zeros((C - 64, D), jnp.float32)]
    ).reshape(C, NCOLCH, NLANES)

    idx = jnp.where(paths_mask.reshape(-1), paths_node_indices.reshape(-1),
                    jnp.int32(DUMP)).astype(jnp.int32)
    idx3 = jnp.concatenate(
        [idx, jnp.full((PLPAD - PL,), DUMP, jnp.int32)]
    ).reshape(NTILES, NCHUNKS, C)

    scattered = _sc_scatter(vals3, tail3, idx3).reshape(N, D)

    Wg_hi = W_gate[:D]
    Wg_lo = W_gate[D:]
    return _tc_update(scattered, previous_nodes_encodings, W_upd, b_upd,
                      Wg_hi, Wg_lo, b_gate)


# speedup vs baseline: 2.6333x; 1.0267x over previous
import functools

import jax
import jax.numpy as jnp
from jax import lax
from jax.experimental import pallas as pl
from jax.experimental.pallas import tpu as pltpu
from jax.experimental.pallas import tpu_sc as plsc

P, L, D, N = 75000, 8, 128, 100000
PL = P * L
NLANES = 16
NCOLCH = D // NLANES
NCORES = 2
NTILES = 16
NPASS = NCOLCH // NCORES
C = 128
G = 4
NCHUNKS = 296
TPOS = NCHUNKS * C
PLPAD = NTILES * TPOS
NG = NCHUNKS // G
T15_FULLG = 61
T15_TAILC = 3
DUMP = N
ZROWS = 100352
ZT = ZROWS // NTILES
ZB = 224
OT = N // NTILES


def _sc_scatter(vals3, tail3, idx3):
    mesh = plsc.VectorSubcoreMesh(core_axis_name="c", subcore_axis_name="s")

    @functools.partial(
        pl.kernel,
        out_type=jax.ShapeDtypeStruct((N, NCOLCH, NLANES), jnp.float32),
        mesh=mesh,
        compiler_params=pltpu.CompilerParams(use_tc_tiling_on_sc=False),
        scratch_types=[
            pltpu.VMEM((2 * G, C), jnp.int32),
            pltpu.VMEM((2 * G * C, 1, NLANES), jnp.float32),
            pltpu.VMEM((ZB, 1, NLANES), jnp.float32),
            pltpu.VMEM_SHARED((ZROWS, 1, NLANES), jnp.float32),
            pltpu.SemaphoreType.DMA((2 * G,)),
            pltpu.SemaphoreType.DMA((2 * G,)),
            pltpu.SemaphoreType.DMA((2,)),
        ],
    )
    def k(vals_hbm, tail_hbm, idx_hbm, out_hbm,
          idxgrp, gbuf, zbuf, acc, gsem, ssem, isem):
        c = lax.axis_index("c")
        s = lax.axis_index("s")
        is_last = s == NTILES - 1
        ngroups = jnp.where(is_last, T15_FULLG, NG)

        def zb_body(i, carry):
            zbuf[i, 0] = jnp.zeros((NLANES,), jnp.float32)
            return carry
        lax.fori_loop(0, ZB, zb_body, 0)

        def idx_desc(g):
            b = lax.rem(g, 2)
            return pltpu.make_async_copy(
                idx_hbm.at[s].at[pl.ds(g * G, G)],
                idxgrp.at[pl.ds(b * G, G)], isem.at[b])

        def gather_desc(kcol, g, j):
            b = lax.rem(g, 2)
            return pltpu.make_async_copy(
                vals_hbm.at[pl.ds(s * TPOS + (g * G + j) * C, C),
                            pl.ds(kcol, 1)],
                gbuf.at[pl.ds((b * G + j) * C, C)], gsem.at[b * G + j])

        def scatter_desc(g, j):
            b = lax.rem(g, 2)
            return pltpu.make_async_copy(
                gbuf.at[pl.ds((b * G + j) * C, C)],
                acc.at[idxgrp.at[b * G + j]], ssem.at[b * G + j])

        for p in range(NPASS):
            kcol = p * NCORES + c

            for j in range(ZT // ZB):
                pltpu.sync_copy(zbuf, acc.at[pl.ds(s * ZT + j * ZB, ZB)])
            plsc.subcore_barrier()

            idx_desc(0).start()
            for j in range(G):
                gather_desc(kcol, 0, j).start()

            def group_body(g, carry):
                for j in range(G):
                    gather_desc(kcol, g, j).wait()
                idx_desc(g).wait()
                for j in range(G):
                    scatter_desc(g, j).start(add=True)
                @pl.when(g > 0)
                def _():
                    for j in range(G):
                        scatter_desc(g - 1, j).wait()

                @pl.when(g + 1 < ngroups)
                def _():
                    idx_desc(g + 1).start()
                    for j in range(G):
                        gather_desc(kcol, g + 1, j).start()
                return carry
            lax.fori_loop(0, ngroups, group_body, 0)
            for j in range(G):
                scatter_desc(ngroups - 1, j).wait()

            @pl.when(is_last)
            def _():
                pltpu.sync_copy(
                    idx_hbm.at[s].at[pl.ds(T15_FULLG * G, G)],
                    idxgrp.at[pl.ds(0, G)])
                for j in range(T15_TAILC):
                    base = s * TPOS + (T15_FULLG * G + j) * C
                    pltpu.sync_copy(
                        vals_hbm.at[pl.ds(base, C), pl.ds(kcol, 1)],
                        gbuf.at[pl.ds(j * C, C)])
                    pltpu.sync_copy(gbuf.at[pl.ds(j * C, C)],
                                    acc.at[idxgrp.at[j]], add=True)
                pltpu.sync_copy(tail_hbm.at[pl.ds(0, C), pl.ds(kcol, 1)],
                                gbuf.at[pl.ds(T15_TAILC * C, C)])
                pltpu.sync_copy(gbuf.at[pl.ds(T15_TAILC * C, C)],
                                acc.at[idxgrp.at[G - 1]], add=True)

            plsc.subcore_barrier()

            pltpu.sync_copy(
                acc.at[pl.ds(s * OT, OT)],
                out_hbm.at[pl.ds(s * OT, OT), pl.ds(kcol, 1)],
            )
            plsc.subcore_barrier()

    return k(vals3, tail3, idx3)


def _tc_update(scattered, prev, W_upd, b_upd, Wg_hi, Wg_lo, b_gate):
    B = 1000
    grid = (N // B,)

    def body(s_ref, p_ref, wu_ref, bu_ref, wgh_ref, wgl_ref, bg_ref, o_ref):
        sblk = s_ref[...]
        prv = p_ref[...]
        upd = jnp.dot(sblk, wu_ref[...], preferred_element_type=jnp.float32)
        upd = jnp.maximum(upd + bu_ref[...], 0.0)
        z = (jnp.dot(prv, wgh_ref[...], preferred_element_type=jnp.float32)
             + jnp.dot(upd, wgl_ref[...], preferred_element_type=jnp.float32)
             + bg_ref[...])
        g = jax.nn.sigmoid(z)
        o_ref[...] = g * prv + (1.0 - g) * upd

    return pl.pallas_call(
        body,
        grid=grid,
        in_specs=[
            pl.BlockSpec((B, D), lambda i: (i, 0)),
            pl.BlockSpec((B, D), lambda i: (i, 0)),
            pl.BlockSpec((D, D), lambda i: (0, 0)),
            pl.BlockSpec((1, D), lambda i: (0, 0)),
            pl.BlockSpec((D, D), lambda i: (0, 0)),
            pl.BlockSpec((D, D), lambda i: (0, 0)),
            pl.BlockSpec((1, D), lambda i: (0, 0)),
        ],
        out_specs=pl.BlockSpec((B, D), lambda i: (i, 0)),
        out_shape=jax.ShapeDtypeStruct((N, D), jnp.float32),
    )(scattered, prev, W_upd, b_upd.reshape(1, D), Wg_hi, Wg_lo,
      b_gate.reshape(1, D))


def kernel(encoded_paths, paths_mask, paths_node_indices,
           previous_nodes_encodings, nr_nodes, W_upd, b_upd, W_gate, b_gate):
    vals = encoded_paths.reshape(PL, D)
    vals3 = vals.reshape(PL, NCOLCH, NLANES)
    tail3 = jnp.concatenate(
        [vals[PL - 64:], jnp.zeros((C - 64, D), jnp.float32)]
    ).reshape(C, NCOLCH, NLANES)

    idx = jnp.where(paths_mask.reshape(-1), paths_node_indices.reshape(-1),
                    jnp.int32(DUMP)).astype(jnp.int32)
    idx3 = jnp.concatenate(
        [idx, jnp.full((PLPAD - PL,), DUMP, jnp.int32)]
    ).reshape(NTILES, NCHUNKS, C)

    scattered = _sc_scatter(vals3, tail3, idx3).reshape(N, D)

    Wg_hi = W_gate[:D]
    Wg_lo = W_gate[D:]
    return _tc_update(scattered, previous_nodes_encodings, W_upd, b_upd,
                      Wg_hi, Wg_lo, b_gate)
